# Initial kernel scaffold; baseline (speedup 1.0000x reference)
import jax
import jax.numpy as jnp
from jax.experimental import pallas as pl


def kernel(input, weights):
    raise NotImplementedError("write your pallas kernel here")


# speedup vs baseline: 1.4647x; 1.4647x over previous
import functools

import jax
import jax.numpy as jnp
from jax import lax
from jax.experimental import pallas as pl
from jax.experimental.pallas import tpu as pltpu
from jax.experimental.pallas import tpu_sc as plsc

_PADDING_IDX = 1
_NC = 2
_NS = 16
_NW = _NC * _NS
_LANES = 16
_CHUNK = 32


@functools.cache
def _build(seq_len: int, bsz: int, vocab: int, dim: int):
    B = seq_len * bsz
    assert B % (_NW * _LANES) == 0 and B % (8 * _NW) == 0
    bpw = B // _NW
    n_chunks = bpw // _CHUNK
    mesh = plsc.VectorSubcoreMesh(core_axis_name="c", subcore_axis_name="s",
                                  num_cores=_NC, num_subcores=_NS)

    @functools.partial(
        pl.kernel,
        out_type=jax.ShapeDtypeStruct((B, dim), jnp.float32),
        mesh=mesh,
        scratch_types=[
            pltpu.VMEM((bpw,), jnp.int32),
            pltpu.VMEM((bpw,), jnp.int32),
            pltpu.VMEM((2, _CHUNK, dim), jnp.float32),
            pltpu.SemaphoreType.DMA,
            pltpu.SemaphoreType.DMA,
        ],
    )
    def k(tok_hbm, w_hbm, out_hbm, tok_v, idx_v, rows_v, sem0, sem1):
        wid = lax.axis_index("s") * _NC + lax.axis_index("c")
        base = wid * bpw
        pltpu.sync_copy(tok_hbm.at[pl.ds(base, bpw)], tok_v)
        for i in range(bpw // _LANES):
            t = tok_v[pl.ds(i * _LANES, _LANES)]
            off = lax.broadcast_in_dim(base + i * _LANES, (_LANES,), ())
            g = off + lax.iota(jnp.int32, _LANES)
            gpos = (g >> bsz.bit_length() - 1 if bsz & (bsz - 1) == 0
                    else g // bsz) + (_PADDING_IDX + 1)
            pos = jnp.where(t != _PADDING_IDX, gpos, t)
            idx_v[pl.ds(i * _LANES, _LANES)] = pos
        sems = (sem0, sem1)
        cps = [None, None]
        cps[0] = pltpu.async_copy(
            w_hbm.at[idx_v.at[pl.ds(0, _CHUNK)]], rows_v.at[0], sems[0])
        for c in range(n_chunks):
            nxt = (c + 1) % 2
            if c + 1 < n_chunks:
                cps[nxt] = pltpu.async_copy(
                    w_hbm.at[idx_v.at[pl.ds((c + 1) * _CHUNK, _CHUNK)]],
                    rows_v.at[nxt], sems[nxt])
            cps[c % 2].wait()
            pltpu.sync_copy(rows_v.at[c % 2],
                            out_hbm.at[pl.ds(base + c * _CHUNK, _CHUNK)])

    return k


def kernel(input, weights):
    seq_len, bsz = input.shape
    vocab, dim = weights.shape
    k = _build(seq_len, bsz, vocab, dim)
    out = k(input.reshape(-1), weights)
    return out.reshape(seq_len, bsz, dim)
